# baseline (speedup 1.0000x reference)
import functools

import jax
import jax.numpy as jnp
from jax import lax
from jax.experimental import pallas as pl
from jax.experimental.pallas import tpu as pltpu
from jax.experimental.pallas import tpu_sc as plsc

NP = 10000
PS = 10
NN = 100000
EP = 300000
EM = 320000
IN = 128
H1 = 512
H2 = 128
RD = 128
HM = 128
OF = 16

PT = 128
NPAD = 10240
NT = NPAD // PT

NWORK = 32
EPW_P = EP // NWORK
STRIDE_P = EPW_P + 1
EPW_M = EM // NWORK
WIN = 128
NWIN_P = -(-EPW_P // WIN)
NWIN_M = -(-EPW_M // WIN)
NWIN_M2 = 80

OFF_A = 0
OFF_PDO = 1_000_000
OFF_PDI = 1_100_000
OFF_MDO = 1_200_000
OFF_MDI = 1_210_000
STOT = 1_220_608

NPR = 10240
CHUNK = 128
NCHUNKP = 80
STRIDE_M = 10240

@functools.cache
def _sc_vector_mesh():
    return plsc.VectorSubcoreMesh(core_axis_name="c", subcore_axis_name="s")


def _leaky(x):
    return jnp.where(x > 0, x, 0.01 * x)


def _sc_stats_body(psrc, pdst, pew, msrc, mdst, zsrc, out, acc, *bufs):
    (sA, dA, wA, sB, dB, wB,
     aiA, oiA, iiA, wuA, ouA, aiB, oiB, iiB, wuB, ouB,
     siA, siB, ssA, ssB) = bufs
    cid = lax.axis_index("c")
    sid = lax.axis_index("s")
    wid = cid * 16 + sid
    seg = STOT // 16

    z0 = sid * seg
    pltpu.sync_copy(zsrc.at[pl.ds(z0, seg)], acc.at[pl.ds(z0, seg)])
    plsc.subcore_barrier()

    def loads(src_hbm, dst_hbm, ew_hbm, base, sb, db, wb, sem):
        pltpu.async_copy(src_hbm.at[pl.ds(base, WIN)], sb, sem)
        pltpu.async_copy(dst_hbm.at[pl.ds(base, WIN)], db, sem)
        if ew_hbm is not None:
            pltpu.async_copy(ew_hbm.at[pl.ds(base, WIN)], wb, sem)

    def wait_loads(src_hbm, sb, db, wb, sem, with_ew):
        n = 3 if with_ew else 2
        for _ in range(n):
            pltpu.make_async_copy(src_hbm.at[pl.ds(0, WIN)], sb, sem).wait()

    def patch_compute(w, sb, db, wb, ai, oi, ii, wu, ou):
        for g in range(WIN // 16):
            sl = pl.ds(g * 16, 16)
            sv = sb[sl]
            dv = db[sl]
            p = ((sv.astype(jnp.float32) + 0.5) * 0.1).astype(jnp.int32)
            j = sv - p * PS
            i = dv - p * PS
            iota = lax.iota(jnp.int32, 16)
            valid = (w * WIN + g * 16 + iota) < EPW_P
            ai[sl] = p * 100 + i * 10 + j
            oi[sl] = OFF_PDO + j * NP + p
            ii[sl] = OFF_PDI + i * NP + p
            wu[sl] = jnp.where(valid, wb[sl], 0.0)
            ou[sl] = jnp.where(valid, 1.0, 0.0)

    def patch_scatters(ai, oi, ii, wu, ou, sem):
        pltpu.async_copy(wu, acc.at[ai], sem, add=True)
        pltpu.async_copy(ou, acc.at[oi], sem, add=True)
        pltpu.async_copy(ou, acc.at[ii], sem, add=True)

    def wait_scatters(ai, wu, sem, n):
        for _ in range(n):
            pltpu.make_async_copy(wu, acc.at[ai], sem).wait()

    base_p = wid * STRIDE_P
    loads(psrc, pdst, pew, base_p, sA, dA, wA, siA)
    loads(psrc, pdst, pew, base_p + WIN, sB, dB, wB, siB)

    def patch_pair(u, _):
        w0 = 2 * u
        w1 = w0 + 1
        wait_loads(psrc, sA, dA, wA, siA, True)
        patch_compute(w0, sA, dA, wA, aiA, oiA, iiA, wuA, ouA)

        @pl.when(u + 1 < NWIN_P // 2)
        def _la():
            loads(psrc, pdst, pew, base_p + (w0 + 2) * WIN, sA, dA, wA, siA)

        @pl.when(u > 0)
        def _wa():
            wait_scatters(aiA, wuA, ssA, 3)
        patch_scatters(aiA, oiA, iiA, wuA, ouA, ssA)

        wait_loads(psrc, sB, dB, wB, siB, True)
        patch_compute(w1, sB, dB, wB, aiB, oiB, iiB, wuB, ouB)

        @pl.when(u + 1 < NWIN_P // 2)
        def _lb():
            loads(psrc, pdst, pew, base_p + (w1 + 2) * WIN, sB, dB, wB, siB)

        @pl.when(u > 0)
        def _wb():
            wait_scatters(aiB, wuB, ssB, 3)
        patch_scatters(aiB, oiB, iiB, wuB, ouB, ssB)
        return _

    lax.fori_loop(0, NWIN_P // 2, patch_pair, 0, unroll=False)
    wait_scatters(aiA, wuA, ssA, 3)
    wait_scatters(aiB, wuB, ssB, 3)

    def mesh_compute(w, sb, db, ai, oi, ou):
        for g in range(WIN // 16):
            sl = pl.ds(g * 16, 16)
            iota = lax.iota(jnp.int32, 16)
            valid = (w * WIN + g * 16 + iota) < EPW_M
            ai[sl] = OFF_MDO + sb[sl]
            oi[sl] = OFF_MDI + db[sl]
            ou[sl] = jnp.where(valid, 1.0, 0.0)

    def mesh_scatters(ai, oi, ou, sem):
        pltpu.async_copy(ou, acc.at[ai], sem, add=True)
        pltpu.async_copy(ou, acc.at[oi], sem, add=True)

    base_m = wid * STRIDE_M
    loads(msrc, mdst, None, base_m, sA, dA, wA, siA)
    loads(msrc, mdst, None, base_m + WIN, sB, dB, wB, siB)

    def mesh_pair(u, _):
        w0 = 2 * u
        w1 = w0 + 1
        wait_loads(msrc, sA, dA, wA, siA, False)
        mesh_compute(w0, sA, dA, aiA, oiA, ouA)

        @pl.when(u + 1 < NWIN_M2 // 2)
        def _la():
            loads(msrc, mdst, None, base_m + (w0 + 2) * WIN, sA, dA, wA, siA)

        @pl.when(u > 0)
        def _wa():
            wait_scatters(aiA, ouA, ssA, 2)
        mesh_scatters(aiA, oiA, ouA, ssA)

        wait_loads(msrc, sB, dB, wB, siB, False)
        mesh_compute(w1, sB, dB, aiB, oiB, ouB)

        @pl.when(u + 1 < NWIN_M2 // 2)
        def _lb():
            loads(msrc, mdst, None, base_m + (w1 + 2) * WIN, sB, dB, wB, siB)

        @pl.when(u > 0)
        def _wb():
            wait_scatters(aiB, ouB, ssB, 2)
        mesh_scatters(aiB, oiB, ouB, ssB)
        return _

    lax.fori_loop(0, NWIN_M2 // 2, mesh_pair, 0, unroll=False)
    wait_scatters(aiA, ouA, ssA, 2)
    wait_scatters(aiB, ouB, ssB, 2)

    plsc.subcore_barrier()
    o = sid * seg
    pltpu.sync_copy(acc.at[pl.ds(o, seg)], out.at[cid, pl.ds(o, seg)])


@jax.jit
def _sc_stats(psrc, pdst, pew, msrc, mdst, zsrc):
    ibuf = lambda: pltpu.VMEM((WIN,), jnp.int32)
    fbuf = lambda: pltpu.VMEM((WIN,), jnp.float32)
    return pl.kernel(
        _sc_stats_body,
        out_type=jax.ShapeDtypeStruct((2, STOT), jnp.float32),
        mesh=_sc_vector_mesh(),
        scratch_types=[
            pltpu.VMEM_SHARED((STOT,), jnp.float32),
            ibuf(), ibuf(), fbuf(), ibuf(), ibuf(), fbuf(),
            ibuf(), ibuf(), ibuf(), fbuf(), fbuf(),
            ibuf(), ibuf(), ibuf(), fbuf(), fbuf(),
            pltpu.SemaphoreType.DMA, pltpu.SemaphoreType.DMA,
            pltpu.SemaphoreType.DMA, pltpu.SemaphoreType.DMA,
        ],
    )(psrc, pdst, pew, msrc, mdst, zsrc)


def _sc_mesh_body(h, msrc, mdst, mew, zsrc, out, acc,
                  srcA, srcB, ewA, ewB, dfA, dfB, drA, drB, rowsA, rowsB,
                  siA, siB, sgA, sgB, ssA, ssB):
    cid = lax.axis_index("c")
    sid = lax.axis_index("s")
    wid = cid * 16 + sid
    rpw = NPR // 16

    r0 = sid * rpw
    pltpu.sync_copy(zsrc.at[pl.ds(r0, rpw)], acc.at[pl.ds(r0, rpw)])
    plsc.subcore_barrier()

    woff = wid * STRIDE_M

    def idx_dmas(t, src_b, ew_b, df_b, sem):
        o = woff + t * CHUNK
        pltpu.async_copy(msrc.at[pl.ds(o, CHUNK)], src_b, sem)
        pltpu.async_copy(mew.at[pl.ds(o, CHUNK)], ew_b, sem)
        pltpu.async_copy(mdst.at[pl.ds(o, CHUNK)], df_b, sem)

    def wait_idx(src_b, ew_b, df_b, sem):
        pltpu.make_async_copy(msrc.at[pl.ds(0, CHUNK)], src_b, sem).wait()
        pltpu.make_async_copy(mew.at[pl.ds(0, CHUNK)], ew_b, sem).wait()
        pltpu.make_async_copy(mdst.at[pl.ds(0, CHUNK)], df_b, sem).wait()

    def scale(ew_b, rows):
        for g in range(CHUNK // 16):
            wv = ew_b[pl.ds(g * 16, 16)]
            for l in range(16):
                e = g * 16 + l
                w = wv[l]
                for k in range(HM // 16):
                    sl = pl.ds(k * 16, 16)
                    rows[e, sl] = rows[e, sl] * w

    def fill_row(df_b, dr_b):
        for g in range(CHUNK // 16):
            sl = pl.ds(g * 16, 16)
            dr_b[0, sl] = df_b[sl]

    idx_dmas(0, srcA, ewA, dfA, siA)
    idx_dmas(1, srcB, ewB, dfB, siB)
    wait_idx(srcA, ewA, dfA, siA)
    pltpu.async_copy(h.at[srcA], rowsA, sgA)
    wait_idx(srcB, ewB, dfB, siB)
    pltpu.async_copy(h.at[srcB], rowsB, sgB)

    def pair(u, _):
        t0 = 2 * u
        t1 = t0 + 1

        pltpu.make_async_copy(h.at[srcA], rowsA, sgA).wait()
        scale(ewA, rowsA)

        @pl.when(u > 0)
        def _wsa():
            pltpu.make_async_copy(rowsA, acc.at[drA.at[0]], ssA).wait()
        fill_row(dfA, drA)
        pltpu.async_copy(rowsA, acc.at[drA.at[0]], ssA, add=True)

        @pl.when(t0 + 2 < NCHUNKP)
        def _ia():
            idx_dmas(t0 + 2, srcA, ewA, dfA, siA)

        pltpu.make_async_copy(h.at[srcB], rowsB, sgB).wait()
        scale(ewB, rowsB)

        @pl.when(u > 0)
        def _wsb():
            pltpu.make_async_copy(rowsB, acc.at[drB.at[0]], ssB).wait()
        fill_row(dfB, drB)
        pltpu.async_copy(rowsB, acc.at[drB.at[0]], ssB, add=True)

        @pl.when(t1 + 2 < NCHUNKP)
        def _ib():
            idx_dmas(t1 + 2, srcB, ewB, dfB, siB)

        @pl.when(t0 + 2 < NCHUNKP)
        def _ga():
            wait_idx(srcA, ewA, dfA, siA)
            pltpu.async_copy(h.at[srcA], rowsA, sgA)

        @pl.when(t1 + 2 < NCHUNKP)
        def _gb():
            wait_idx(srcB, ewB, dfB, siB)
            pltpu.async_copy(h.at[srcB], rowsB, sgB)

        return _

    lax.fori_loop(0, NCHUNKP // 2, pair, 0, unroll=False)

    pltpu.make_async_copy(rowsA, acc.at[drA.at[0]], ssA).wait()
    pltpu.make_async_copy(rowsB, acc.at[drB.at[0]], ssB).wait()

    plsc.subcore_barrier()
    pltpu.sync_copy(acc.at[pl.ds(r0, rpw)], out.at[cid, pl.ds(r0, rpw)])


@jax.jit
def _sc_mesh(h, msrc, mdst, mew, zsrc):
    return pl.kernel(
        _sc_mesh_body,
        out_type=jax.ShapeDtypeStruct((2, NPR, HM), jnp.float32),
        mesh=_sc_vector_mesh(),
        scratch_types=[
            pltpu.VMEM_SHARED((NPR, HM), jnp.float32),
            pltpu.VMEM((CHUNK,), jnp.int32),
            pltpu.VMEM((CHUNK,), jnp.int32),
            pltpu.VMEM((CHUNK,), jnp.float32),
            pltpu.VMEM((CHUNK,), jnp.float32),
            pltpu.VMEM((CHUNK,), jnp.int32),
            pltpu.VMEM((CHUNK,), jnp.int32),
            pltpu.VMEM((1, CHUNK), jnp.int32),
            pltpu.VMEM((1, CHUNK), jnp.int32),
            pltpu.VMEM((CHUNK, HM), jnp.float32),
            pltpu.VMEM((CHUNK, HM), jnp.float32),
            pltpu.SemaphoreType.DMA,
            pltpu.SemaphoreType.DMA,
            pltpu.SemaphoreType.DMA,
            pltpu.SemaphoreType.DMA,
            pltpu.SemaphoreType.DMA,
            pltpu.SemaphoreType.DMA,
        ],
    )(h, msrc, mdst, mew, zsrc)


def _tc_patch_body(xs_ref, a_ref, pdo_ref, pdi_ref, mdo_ref,
                   w1t_ref, w2t_ref, wet_ref, wm1t_ref,
                   a1_ref, g1_ref, b1_ref, a2_ref, g2_ref, b2_ref,
                   ro_ref, hp_ref):
    f32 = jnp.float32
    A2 = a_ref[0] + a_ref[1]
    do = lax.rsqrt(jnp.maximum(pdo_ref[0] + pdo_ref[1], 1.0))
    di = lax.rsqrt(jnp.maximum(pdi_ref[0] + pdi_ref[1], 1.0))
    xs = [xs_ref[j] for j in range(PS)]

    r0 = sum(xs) * (1.0 / PS)

    wA = [[A2[i * PS + j:i * PS + j + 1, :] * do[j:j + 1, :]
           for j in range(PS)] for i in range(PS)]

    def mix(vs):
        outs = []
        for i in range(PS):
            acc = wA[i][0] * vs[0]
            for j in range(1, PS):
                acc = acc + wA[i][j] * vs[j]
            outs.append(di[i:i + 1, :] * acc)
        return outs

    def gnorm(hs, ac, gc, bc):
        m = sum(hs) * (1.0 / PS)
        sub = [h - ac * m for h in hs]
        var = sum(s * s for s in sub) * (1.0 / PS)
        inv = lax.rsqrt(var + 1e-5)
        return [_leaky(gc * s * inv + bc) for s in sub]

    mixed = mix(xs)
    h1 = [jnp.dot(w1t_ref[...], m, preferred_element_type=f32) for m in mixed]
    h1n = gnorm(h1, a1_ref[:, 0:1], g1_ref[:, 0:1], b1_ref[:, 0:1])
    r1 = sum(h1n) * (1.0 / PS)

    t = [jnp.dot(w2t_ref[...], h, preferred_element_type=f32) for h in h1n]
    mixed2 = mix(t)
    h2n = gnorm(mixed2, a2_ref[:, 0:1], g2_ref[:, 0:1], b2_ref[:, 0:1])
    r2 = sum(h2n) * (1.0 / PS)

    cat = jnp.concatenate([r0, r1, r2], axis=0)
    emb = jnp.dot(wet_ref[...], cat, preferred_element_type=f32)
    mu = jnp.mean(emb, axis=0, keepdims=True)
    var = jnp.mean((emb - mu) ** 2, axis=0, keepdims=True)
    ro = _leaky((emb - mu) * lax.rsqrt(var + 1e-5))
    ro_ref[...] = ro

    mdo_r = lax.rsqrt(jnp.maximum(mdo_ref[0, 0:1, :] + mdo_ref[1, 0:1, :], 1.0))
    hp_ref[...] = (jnp.dot(wm1t_ref[...], ro, preferred_element_type=f32)
                   * mdo_r).T


@jax.jit
def _tc_patch(xs, a_pad, pdo, pdi, mdo, w1t, w2t, wet, wm1t,
              a1c, g1c, b1c, a2c, g2c, b2c):
    full = lambda s: pl.BlockSpec(s, lambda t: (0,) * len(s))
    tiled3 = lambda d0, d1: pl.BlockSpec((d0, d1, PT), lambda t: (0, 0, t))
    return pl.pallas_call(
        _tc_patch_body,
        grid=(NT,),
        in_specs=[
            tiled3(PS, IN),
            tiled3(2, 104),
            tiled3(2, 16),
            tiled3(2, 16),
            tiled3(2, 8),
            full((H1, IN)), full((H2, H1)), full((RD, IN + H1 + H2)),
            full((HM, RD)),
            full((H1, 8)), full((H1, 8)), full((H1, 8)),
            full((H2, 8)), full((H2, 8)), full((H2, 8)),
        ],
        out_specs=[
            pl.BlockSpec((RD, PT), lambda t: (0, t)),
            pl.BlockSpec((PT, HM), lambda t: (t, 0)),
        ],
        out_shape=[
            jax.ShapeDtypeStruct((RD, NPAD), jnp.float32),
            jax.ShapeDtypeStruct((NPAD, HM), jnp.float32),
        ],
        compiler_params=pltpu.CompilerParams(vmem_limit_bytes=100 * 1024 * 1024),
    )(xs, a_pad, pdo, pdi, mdo, w1t, w2t, wet, wm1t,
      a1c, g1c, b1c, a2c, g2c, b2c)


def _mesh_norm(x, ar, gr, br):
    mu = jnp.mean(x, axis=0, keepdims=True)
    sub = x - ar * mu
    var = jnp.mean(sub * sub, axis=0, keepdims=True)
    return _leaky(gr * sub * lax.rsqrt(var + 1e-5) + br)


def _tc_mesh1_body(agg_ref, mdi_ref, mdo_ref, wm2_ref, am_ref, gm_ref, bm_ref,
                   hp_ref, ra_ref):
    mdi = lax.rsqrt(jnp.maximum(mdi_ref[:, 0:1] + mdi_ref[:, 1:2], 1.0))
    x = (agg_ref[0] + agg_ref[1]) * mdi
    hm = _mesh_norm(x, am_ref[0:1, :], gm_ref[0:1, :], bm_ref[0:1, :])
    ra_ref[...] = jnp.broadcast_to(jnp.mean(hm, axis=0, keepdims=True), (8, HM))
    mdo = lax.rsqrt(jnp.maximum(mdo_ref[:, 0:1] + mdo_ref[:, 1:2], 1.0))
    hp_ref[...] = jnp.dot(hm, wm2_ref[...],
                          preferred_element_type=jnp.float32) * mdo


@jax.jit
def _tc_mesh1(agg, mdi, mdo, wm2, amr, gmr, bmr):
    full = lambda s: pl.BlockSpec(s, lambda: (0,) * len(s))
    return pl.pallas_call(
        _tc_mesh1_body,
        in_specs=[full((2, NP, HM)), full((NP, 8)), full((NP, 8)),
                  full((HM, HM)), full((8, HM)), full((8, HM)), full((8, HM))],
        out_specs=[full((NP, HM)), full((8, HM))],
        out_shape=[jax.ShapeDtypeStruct((NP, HM), jnp.float32),
                   jax.ShapeDtypeStruct((8, HM), jnp.float32)],
        compiler_params=pltpu.CompilerParams(vmem_limit_bytes=100 * 1024 * 1024),
    )(agg, mdi, mdo, wm2, amr, gmr, bmr)


def _tc_mesh2_body(agg_ref, mdi_ref, am_ref, gm_ref, bm_ref, ra_ref, wc_ref,
                   out_ref):
    mdi = lax.rsqrt(jnp.maximum(mdi_ref[:, 0:1] + mdi_ref[:, 1:2], 1.0))
    x = (agg_ref[0] + agg_ref[1]) * mdi
    hm = _mesh_norm(x, am_ref[0:1, :], gm_ref[0:1, :], bm_ref[0:1, :])
    rb = jnp.mean(hm, axis=0, keepdims=True)
    cat = jnp.concatenate([ra_ref[0:1, :], rb], axis=1)
    blk = jnp.broadcast_to(_leaky(cat), (8, 2 * HM))
    out_ref[...] = jnp.dot(blk, wc_ref[...], preferred_element_type=jnp.float32)


@jax.jit
def _tc_mesh2(agg, mdi, amr, gmr, bmr, ra, wc):
    full = lambda s: pl.BlockSpec(s, lambda: (0,) * len(s))
    return pl.pallas_call(
        _tc_mesh2_body,
        in_specs=[full((2, NP, HM)), full((NP, 8)),
                  full((8, HM)), full((8, HM)), full((8, HM)), full((8, HM)),
                  full((2 * HM, OF))],
        out_specs=full((8, OF)),
        out_shape=jax.ShapeDtypeStruct((8, OF), jnp.float32),
        compiler_params=pltpu.CompilerParams(vmem_limit_bytes=100 * 1024 * 1024),
    )(agg, mdi, amr, gmr, bmr, ra, wc)


def kernel(patch_feats, patch_ew, mesh_ew, W1, a1, g1, b1, W2, a2, g2, b2, We,
           Wm1, am1, gm1, bm1, Wm2, am2, gm2, bm2, Wc,
           patch_src, patch_dst, patch_seg, mesh_src, mesh_dst):
    i32 = jnp.int32
    f32 = jnp.float32

    restride = lambda v: jnp.pad(
        jnp.pad(v.reshape(NWORK, EPW_P), ((0, 0), (0, 1))).reshape(-1), (0, WIN))
    psrc = restride(patch_src.astype(i32))
    pdst = restride(patch_dst.astype(i32))
    pew = restride(patch_ew.astype(f32))
    restride_m = lambda v: jnp.pad(
        v.reshape(NWORK, EPW_M), ((0, 0), (0, STRIDE_M - EPW_M))).reshape(-1)
    msrc = restride_m(mesh_src.astype(i32))
    mdst = restride_m(mesh_dst.astype(i32))
    mewp = restride_m(mesh_ew.astype(f32))
    z1 = jnp.zeros((STOT,), f32)
    z2 = jnp.zeros((NPR, HM), f32)

    stats = _sc_stats(psrc, pdst, pew, msrc, mdst, z1)

    a_pad = jnp.pad(stats[:, OFF_A:OFF_A + 100 * NP].reshape(2, 100, NP),
                    ((0, 0), (0, 4), (0, NPAD - NP)))
    pdo = jnp.pad(stats[:, OFF_PDO:OFF_PDO + PS * NP].reshape(2, PS, NP),
                  ((0, 0), (0, 16 - PS), (0, NPAD - NP)))
    pdi = jnp.pad(stats[:, OFF_PDI:OFF_PDI + PS * NP].reshape(2, PS, NP),
                  ((0, 0), (0, 16 - PS), (0, NPAD - NP)))
    mdo_raw = stats[:, OFF_MDO:OFF_MDO + NP]
    mdi_raw = stats[:, OFF_MDI:OFF_MDI + NP]
    mdo_p = jnp.pad(mdo_raw[:, None, :], ((0, 0), (0, 7), (0, NPAD - NP)))
    mdo_m = jnp.pad(mdo_raw[:, None, :], ((0, 0), (0, 7), (0, 0)))
    mdi_m = jnp.pad(mdi_raw[:, None, :], ((0, 0), (0, 7), (0, 0)))

    xs = jnp.pad(patch_feats.reshape(NP, PS, IN).transpose(1, 2, 0),
                 ((0, 0), (0, 0), (0, NPAD - NP)))
    col = lambda v: jnp.broadcast_to(v[:, None], (v.shape[0], 8))

    roT, hp1T = _tc_patch(xs, a_pad, pdo, pdi, mdo_p,
                          W1.T, W2.T, We.T, Wm1.T,
                          col(a1), col(g1), col(b1),
                          col(a2), col(g2), col(b2))

    mdi_c = jnp.pad(mdi_raw.T, ((0, 0), (0, 6)))
    mdo_c = jnp.pad(mdo_raw.T, ((0, 0), (0, 6)))
    row = lambda v: jnp.broadcast_to(v[None, :], (8, v.shape[0]))

    hp1 = hp1T[:NP]
    agg1 = _sc_mesh(hp1, msrc, mdst, mewp, z2)[:, :NP]

    hp2, ra = _tc_mesh1(agg1, mdi_c, mdo_c, Wm2,
                        row(am1), row(gm1), row(bm1))

    agg2 = _sc_mesh(hp2, msrc, mdst, mewp, z2)[:, :NP]

    out = _tc_mesh2(agg2, mdi_c, row(am2), row(gm2), row(bm2), ra, Wc)
    return out[0:1, :]


# speedup vs baseline: 1.0023x; 1.0023x over previous
import functools

import jax
import jax.numpy as jnp
from jax import lax
from jax.experimental import pallas as pl
from jax.experimental.pallas import tpu as pltpu
from jax.experimental.pallas import tpu_sc as plsc

NP = 10000
PS = 10
NN = 100000
EP = 300000
EM = 320000
IN = 128
H1 = 512
H2 = 128
RD = 128
HM = 128
OF = 16

PT = 128
NPAD = 10240
NT = NPAD // PT

NWORK = 32
EPW_P = EP // NWORK
STRIDE_P = EPW_P + 1
EPW_M = EM // NWORK
WIN = 128
NWIN_P = -(-EPW_P // WIN)
NWIN_M = -(-EPW_M // WIN)
NWIN_M2 = 80

OFF_A = 0
OFF_PDO = 1_000_000
OFF_PDI = 1_100_000
OFF_MDO = 1_200_000
OFF_MDI = 1_210_000
STOT = 1_220_608

NPR = 10240
CHUNK = 128
NCHUNKP = 80
STRIDE_M = 10240

@functools.cache
def _sc_vector_mesh():
    return plsc.VectorSubcoreMesh(core_axis_name="c", subcore_axis_name="s")


def _leaky(x):
    return jnp.where(x > 0, x, 0.01 * x)


def _sc_stats_body(psrc, pdst, pew, msrc, mdst, zsrc, out, acc, *bufs):
    (sA, dA, wA, sB, dB, wB,
     aiA, oiA, iiA, wuA, ouA, aiB, oiB, iiB, wuB, ouB,
     siA, siB, ssA, ssB) = bufs
    cid = lax.axis_index("c")
    sid = lax.axis_index("s")
    wid = cid * 16 + sid
    seg = STOT // 16

    z0 = sid * seg
    pltpu.sync_copy(zsrc.at[pl.ds(z0, seg)], acc.at[pl.ds(z0, seg)])
    plsc.subcore_barrier()

    def loads(src_hbm, dst_hbm, ew_hbm, base, sb, db, wb, sem):
        pltpu.async_copy(src_hbm.at[pl.ds(base, WIN)], sb, sem)
        pltpu.async_copy(dst_hbm.at[pl.ds(base, WIN)], db, sem)
        if ew_hbm is not None:
            pltpu.async_copy(ew_hbm.at[pl.ds(base, WIN)], wb, sem)

    def wait_loads(src_hbm, sb, db, wb, sem, with_ew):
        n = 3 if with_ew else 2
        for _ in range(n):
            pltpu.make_async_copy(src_hbm.at[pl.ds(0, WIN)], sb, sem).wait()

    def patch_compute(w, sb, db, wb, ai, oi, ii, wu, ou):
        for g in range(WIN // 16):
            sl = pl.ds(g * 16, 16)
            sv = sb[sl]
            dv = db[sl]
            p = ((sv.astype(jnp.float32) + 0.5) * 0.1).astype(jnp.int32)
            j = sv - p * PS
            i = dv - p * PS
            iota = lax.iota(jnp.int32, 16)
            valid = (w * WIN + g * 16 + iota) < EPW_P
            ai[sl] = p * 100 + i * 10 + j
            oi[sl] = OFF_PDO + j * NP + p
            ii[sl] = OFF_PDI + i * NP + p
            wu[sl] = jnp.where(valid, wb[sl], 0.0)
            ou[sl] = jnp.where(valid, 1.0, 0.0)

    def patch_scatters(ai, oi, ii, wu, ou, sem):
        pltpu.async_copy(wu, acc.at[ai], sem, add=True)
        pltpu.async_copy(ou, acc.at[oi], sem, add=True)
        pltpu.async_copy(ou, acc.at[ii], sem, add=True)

    def wait_scatters(ai, wu, sem, n):
        for _ in range(n):
            pltpu.make_async_copy(wu, acc.at[ai], sem).wait()

    base_p = wid * STRIDE_P
    loads(psrc, pdst, pew, base_p, sA, dA, wA, siA)
    loads(psrc, pdst, pew, base_p + WIN, sB, dB, wB, siB)

    def patch_pair(u, _):
        w0 = 2 * u
        w1 = w0 + 1
        wait_loads(psrc, sA, dA, wA, siA, True)
        patch_compute(w0, sA, dA, wA, aiA, oiA, iiA, wuA, ouA)

        @pl.when(u + 1 < NWIN_P // 2)
        def _la():
            loads(psrc, pdst, pew, base_p + (w0 + 2) * WIN, sA, dA, wA, siA)

        @pl.when(u > 0)
        def _wa():
            wait_scatters(aiA, wuA, ssA, 3)
        patch_scatters(aiA, oiA, iiA, wuA, ouA, ssA)

        wait_loads(psrc, sB, dB, wB, siB, True)
        patch_compute(w1, sB, dB, wB, aiB, oiB, iiB, wuB, ouB)

        @pl.when(u + 1 < NWIN_P // 2)
        def _lb():
            loads(psrc, pdst, pew, base_p + (w1 + 2) * WIN, sB, dB, wB, siB)

        @pl.when(u > 0)
        def _wb():
            wait_scatters(aiB, wuB, ssB, 3)
        patch_scatters(aiB, oiB, iiB, wuB, ouB, ssB)
        return _

    lax.fori_loop(0, NWIN_P // 2, patch_pair, 0, unroll=False)
    wait_scatters(aiA, wuA, ssA, 3)
    wait_scatters(aiB, wuB, ssB, 3)

    def mesh_compute(w, sb, db, ai, oi, ou):
        for g in range(WIN // 16):
            sl = pl.ds(g * 16, 16)
            iota = lax.iota(jnp.int32, 16)
            valid = (w * WIN + g * 16 + iota) < EPW_M
            ai[sl] = OFF_MDO + sb[sl]
            oi[sl] = OFF_MDI + db[sl]
            ou[sl] = jnp.where(valid, 1.0, 0.0)

    def mesh_scatters(ai, oi, ou, sem):
        pltpu.async_copy(ou, acc.at[ai], sem, add=True)
        pltpu.async_copy(ou, acc.at[oi], sem, add=True)

    base_m = wid * STRIDE_M
    loads(msrc, mdst, None, base_m, sA, dA, wA, siA)
    loads(msrc, mdst, None, base_m + WIN, sB, dB, wB, siB)

    def mesh_pair(u, _):
        w0 = 2 * u
        w1 = w0 + 1
        wait_loads(msrc, sA, dA, wA, siA, False)
        mesh_compute(w0, sA, dA, aiA, oiA, ouA)

        @pl.when(u + 1 < NWIN_M2 // 2)
        def _la():
            loads(msrc, mdst, None, base_m + (w0 + 2) * WIN, sA, dA, wA, siA)

        @pl.when(u > 0)
        def _wa():
            wait_scatters(aiA, ouA, ssA, 2)
        mesh_scatters(aiA, oiA, ouA, ssA)

        wait_loads(msrc, sB, dB, wB, siB, False)
        mesh_compute(w1, sB, dB, aiB, oiB, ouB)

        @pl.when(u + 1 < NWIN_M2 // 2)
        def _lb():
            loads(msrc, mdst, None, base_m + (w1 + 2) * WIN, sB, dB, wB, siB)

        @pl.when(u > 0)
        def _wb():
            wait_scatters(aiB, ouB, ssB, 2)
        mesh_scatters(aiB, oiB, ouB, ssB)
        return _

    lax.fori_loop(0, NWIN_M2 // 2, mesh_pair, 0, unroll=False)
    wait_scatters(aiA, ouA, ssA, 2)
    wait_scatters(aiB, ouB, ssB, 2)

    plsc.subcore_barrier()
    o = sid * seg
    pltpu.sync_copy(acc.at[pl.ds(o, seg)], out.at[cid, pl.ds(o, seg)])


@jax.jit
def _sc_stats(psrc, pdst, pew, msrc, mdst, zsrc):
    ibuf = lambda: pltpu.VMEM((WIN,), jnp.int32)
    fbuf = lambda: pltpu.VMEM((WIN,), jnp.float32)
    return pl.kernel(
        _sc_stats_body,
        out_type=jax.ShapeDtypeStruct((2, STOT), jnp.float32),
        mesh=_sc_vector_mesh(),
        scratch_types=[
            pltpu.VMEM_SHARED((STOT,), jnp.float32),
            ibuf(), ibuf(), fbuf(), ibuf(), ibuf(), fbuf(),
            ibuf(), ibuf(), ibuf(), fbuf(), fbuf(),
            ibuf(), ibuf(), ibuf(), fbuf(), fbuf(),
            pltpu.SemaphoreType.DMA, pltpu.SemaphoreType.DMA,
            pltpu.SemaphoreType.DMA, pltpu.SemaphoreType.DMA,
        ],
    )(psrc, pdst, pew, msrc, mdst, zsrc)


def _sc_mesh_body(h, msrc, mdst, mew, zsrc, out, acc,
                  srcA, srcB, ewA, ewB, dfA, dfB, drA, drB, rowsA, rowsB,
                  siA, siB, sgA, sgB, ssA, ssB):
    cid = lax.axis_index("c")
    sid = lax.axis_index("s")
    wid = cid * 16 + sid
    rpw = NPR // 16

    r0 = sid * rpw
    pltpu.sync_copy(zsrc.at[pl.ds(r0, rpw)], acc.at[pl.ds(r0, rpw)])
    plsc.subcore_barrier()

    woff = wid * STRIDE_M

    def idx_dmas(t, src_b, ew_b, df_b, sem):
        o = woff + t * CHUNK
        pltpu.async_copy(msrc.at[pl.ds(o, CHUNK)], src_b, sem)
        pltpu.async_copy(mew.at[pl.ds(o, CHUNK)], ew_b, sem)
        pltpu.async_copy(mdst.at[pl.ds(o, CHUNK)], df_b, sem)

    def wait_idx(src_b, ew_b, df_b, sem):
        pltpu.make_async_copy(msrc.at[pl.ds(0, CHUNK)], src_b, sem).wait()
        pltpu.make_async_copy(mew.at[pl.ds(0, CHUNK)], ew_b, sem).wait()
        pltpu.make_async_copy(mdst.at[pl.ds(0, CHUNK)], df_b, sem).wait()

    def scale(ew_b, rows):
        for g in range(CHUNK // 16):
            wv = ew_b[pl.ds(g * 16, 16)]
            for l in range(16):
                e = g * 16 + l
                w = wv[l]
                for k in range(HM // 16):
                    sl = pl.ds(k * 16, 16)
                    rows[e, sl] = rows[e, sl] * w

    def fill_row(df_b, dr_b):
        for g in range(CHUNK // 16):
            sl = pl.ds(g * 16, 16)
            dr_b[0, sl] = df_b[sl]

    idx_dmas(0, srcA, ewA, dfA, siA)
    idx_dmas(1, srcB, ewB, dfB, siB)
    wait_idx(srcA, ewA, dfA, siA)
    pltpu.async_copy(h.at[srcA], rowsA, sgA)
    wait_idx(srcB, ewB, dfB, siB)
    pltpu.async_copy(h.at[srcB], rowsB, sgB)

    def pair(u, _):
        t0 = 2 * u
        t1 = t0 + 1

        pltpu.make_async_copy(h.at[srcA], rowsA, sgA).wait()
        scale(ewA, rowsA)

        @pl.when(u > 0)
        def _wsa():
            pltpu.make_async_copy(rowsA, acc.at[drA.at[0]], ssA).wait()
        fill_row(dfA, drA)
        pltpu.async_copy(rowsA, acc.at[drA.at[0]], ssA, add=True)

        @pl.when(t0 + 2 < NCHUNKP)
        def _ia():
            idx_dmas(t0 + 2, srcA, ewA, dfA, siA)

        pltpu.make_async_copy(h.at[srcB], rowsB, sgB).wait()
        scale(ewB, rowsB)

        @pl.when(u > 0)
        def _wsb():
            pltpu.make_async_copy(rowsB, acc.at[drB.at[0]], ssB).wait()
        fill_row(dfB, drB)
        pltpu.async_copy(rowsB, acc.at[drB.at[0]], ssB, add=True)

        @pl.when(t1 + 2 < NCHUNKP)
        def _ib():
            idx_dmas(t1 + 2, srcB, ewB, dfB, siB)

        @pl.when(t0 + 2 < NCHUNKP)
        def _ga():
            wait_idx(srcA, ewA, dfA, siA)
            pltpu.async_copy(h.at[srcA], rowsA, sgA)

        @pl.when(t1 + 2 < NCHUNKP)
        def _gb():
            wait_idx(srcB, ewB, dfB, siB)
            pltpu.async_copy(h.at[srcB], rowsB, sgB)

        return _

    lax.fori_loop(0, NCHUNKP // 2, pair, 0, unroll=False)

    pltpu.make_async_copy(rowsA, acc.at[drA.at[0]], ssA).wait()
    pltpu.make_async_copy(rowsB, acc.at[drB.at[0]], ssB).wait()

    plsc.subcore_barrier()
    pltpu.sync_copy(acc.at[pl.ds(r0, rpw)], out.at[cid, pl.ds(r0, rpw)])


@jax.jit
def _sc_mesh(h, msrc, mdst, mew, zsrc):
    return pl.kernel(
        _sc_mesh_body,
        out_type=jax.ShapeDtypeStruct((2, NPR, HM), jnp.float32),
        mesh=_sc_vector_mesh(),
        scratch_types=[
            pltpu.VMEM_SHARED((NPR, HM), jnp.float32),
            pltpu.VMEM((CHUNK,), jnp.int32),
            pltpu.VMEM((CHUNK,), jnp.int32),
            pltpu.VMEM((CHUNK,), jnp.float32),
            pltpu.VMEM((CHUNK,), jnp.float32),
            pltpu.VMEM((CHUNK,), jnp.int32),
            pltpu.VMEM((CHUNK,), jnp.int32),
            pltpu.VMEM((1, CHUNK), jnp.int32),
            pltpu.VMEM((1, CHUNK), jnp.int32),
            pltpu.VMEM((CHUNK, HM), jnp.float32),
            pltpu.VMEM((CHUNK, HM), jnp.float32),
            pltpu.SemaphoreType.DMA,
            pltpu.SemaphoreType.DMA,
            pltpu.SemaphoreType.DMA,
            pltpu.SemaphoreType.DMA,
            pltpu.SemaphoreType.DMA,
            pltpu.SemaphoreType.DMA,
        ],
    )(h, msrc, mdst, mew, zsrc)


def _tc_patch_body(xs_ref, a_ref, pdo_ref, pdi_ref, mdo_ref,
                   w1t_ref, w2t_ref, wet_ref, wm1t_ref,
                   a1_ref, g1_ref, b1_ref, a2_ref, g2_ref, b2_ref,
                   ro_ref, hp_ref):
    f32 = jnp.float32
    A2 = a_ref[0] + a_ref[1]
    do = lax.rsqrt(jnp.maximum(pdo_ref[0] + pdo_ref[1], 1.0))
    di = lax.rsqrt(jnp.maximum(pdi_ref[0] + pdi_ref[1], 1.0))
    xs = [xs_ref[j] for j in range(PS)]

    r0 = sum(xs) * (1.0 / PS)

    wA = [[A2[i * PS + j:i * PS + j + 1, :] * do[j:j + 1, :]
           for j in range(PS)] for i in range(PS)]

    def mix(vs):
        outs = []
        for i in range(PS):
            acc = wA[i][0] * vs[0]
            for j in range(1, PS):
                acc = acc + wA[i][j] * vs[j]
            outs.append(di[i:i + 1, :] * acc)
        return outs

    def gnorm(hs, ac, gc, bc):
        m = sum(hs) * (1.0 / PS)
        sub = [h - ac * m for h in hs]
        var = sum(s * s for s in sub) * (1.0 / PS)
        inv = lax.rsqrt(var + 1e-5)
        return [_leaky(gc * s * inv + bc) for s in sub]

    mixed = mix(xs)
    h1 = [jnp.dot(w1t_ref[...], m, preferred_element_type=f32) for m in mixed]
    h1n = gnorm(h1, a1_ref[:, 0:1], g1_ref[:, 0:1], b1_ref[:, 0:1])
    r1 = sum(h1n) * (1.0 / PS)

    t = [jnp.dot(w2t_ref[...], h, preferred_element_type=f32) for h in h1n]
    mixed2 = mix(t)
    h2n = gnorm(mixed2, a2_ref[:, 0:1], g2_ref[:, 0:1], b2_ref[:, 0:1])
    r2 = sum(h2n) * (1.0 / PS)

    cat = jnp.concatenate([r0, r1, r2], axis=0)
    emb = jnp.dot(wet_ref[...], cat, preferred_element_type=f32)
    mu = jnp.mean(emb, axis=0, keepdims=True)
    var = jnp.mean((emb - mu) ** 2, axis=0, keepdims=True)
    ro = _leaky((emb - mu) * lax.rsqrt(var + 1e-5))
    ro_ref[...] = ro

    mdo_r = lax.rsqrt(jnp.maximum(mdo_ref[0, 0:1, :] + mdo_ref[1, 0:1, :], 1.0))
    hp_ref[...] = jnp.dot(wm1t_ref[...], ro, preferred_element_type=f32) * mdo_r


@jax.jit
def _tc_patch(xs, a_pad, pdo, pdi, mdo, w1t, w2t, wet, wm1t,
              a1c, g1c, b1c, a2c, g2c, b2c):
    full = lambda s: pl.BlockSpec(s, lambda t: (0,) * len(s))
    tiled3 = lambda d0, d1: pl.BlockSpec((d0, d1, PT), lambda t: (0, 0, t))
    return pl.pallas_call(
        _tc_patch_body,
        grid=(NT,),
        in_specs=[
            tiled3(PS, IN),
            tiled3(2, 104),
            tiled3(2, 16),
            tiled3(2, 16),
            tiled3(2, 8),
            full((H1, IN)), full((H2, H1)), full((RD, IN + H1 + H2)),
            full((HM, RD)),
            full((H1, 8)), full((H1, 8)), full((H1, 8)),
            full((H2, 8)), full((H2, 8)), full((H2, 8)),
        ],
        out_specs=[
            pl.BlockSpec((RD, PT), lambda t: (0, t)),
            pl.BlockSpec((HM, PT), lambda t: (0, t)),
        ],
        out_shape=[
            jax.ShapeDtypeStruct((RD, NPAD), jnp.float32),
            jax.ShapeDtypeStruct((HM, NPAD), jnp.float32),
        ],
        compiler_params=pltpu.CompilerParams(vmem_limit_bytes=100 * 1024 * 1024),
    )(xs, a_pad, pdo, pdi, mdo, w1t, w2t, wet, wm1t,
      a1c, g1c, b1c, a2c, g2c, b2c)


def _mesh_norm(x, ar, gr, br):
    mu = jnp.mean(x, axis=0, keepdims=True)
    sub = x - ar * mu
    var = jnp.mean(sub * sub, axis=0, keepdims=True)
    return _leaky(gr * sub * lax.rsqrt(var + 1e-5) + br)


def _tc_mesh1_body(agg_ref, mdi_ref, mdo_ref, wm2_ref, am_ref, gm_ref, bm_ref,
                   hp_ref, ra_ref):
    mdi = lax.rsqrt(jnp.maximum(mdi_ref[:, 0:1] + mdi_ref[:, 1:2], 1.0))
    x = (agg_ref[0] + agg_ref[1]) * mdi
    hm = _mesh_norm(x, am_ref[0:1, :], gm_ref[0:1, :], bm_ref[0:1, :])
    ra_ref[...] = jnp.broadcast_to(jnp.mean(hm, axis=0, keepdims=True), (8, HM))
    mdo = lax.rsqrt(jnp.maximum(mdo_ref[:, 0:1] + mdo_ref[:, 1:2], 1.0))
    hp_ref[...] = jnp.dot(hm, wm2_ref[...],
                          preferred_element_type=jnp.float32) * mdo


@jax.jit
def _tc_mesh1(agg, mdi, mdo, wm2, amr, gmr, bmr):
    full = lambda s: pl.BlockSpec(s, lambda: (0,) * len(s))
    return pl.pallas_call(
        _tc_mesh1_body,
        in_specs=[full((2, NP, HM)), full((NP, 8)), full((NP, 8)),
                  full((HM, HM)), full((8, HM)), full((8, HM)), full((8, HM))],
        out_specs=[full((NP, HM)), full((8, HM))],
        out_shape=[jax.ShapeDtypeStruct((NP, HM), jnp.float32),
                   jax.ShapeDtypeStruct((8, HM), jnp.float32)],
        compiler_params=pltpu.CompilerParams(vmem_limit_bytes=100 * 1024 * 1024),
    )(agg, mdi, mdo, wm2, amr, gmr, bmr)


def _tc_mesh2_body(agg_ref, mdi_ref, am_ref, gm_ref, bm_ref, ra_ref, wc_ref,
                   out_ref):
    mdi = lax.rsqrt(jnp.maximum(mdi_ref[:, 0:1] + mdi_ref[:, 1:2], 1.0))
    x = (agg_ref[0] + agg_ref[1]) * mdi
    hm = _mesh_norm(x, am_ref[0:1, :], gm_ref[0:1, :], bm_ref[0:1, :])
    rb = jnp.mean(hm, axis=0, keepdims=True)
    cat = jnp.concatenate([ra_ref[0:1, :], rb], axis=1)
    blk = jnp.broadcast_to(_leaky(cat), (8, 2 * HM))
    out_ref[...] = jnp.dot(blk, wc_ref[...], preferred_element_type=jnp.float32)


@jax.jit
def _tc_mesh2(agg, mdi, amr, gmr, bmr, ra, wc):
    full = lambda s: pl.BlockSpec(s, lambda: (0,) * len(s))
    return pl.pallas_call(
        _tc_mesh2_body,
        in_specs=[full((2, NP, HM)), full((NP, 8)),
                  full((8, HM)), full((8, HM)), full((8, HM)), full((8, HM)),
                  full((2 * HM, OF))],
        out_specs=full((8, OF)),
        out_shape=jax.ShapeDtypeStruct((8, OF), jnp.float32),
        compiler_params=pltpu.CompilerParams(vmem_limit_bytes=100 * 1024 * 1024),
    )(agg, mdi, amr, gmr, bmr, ra, wc)


def kernel(patch_feats, patch_ew, mesh_ew, W1, a1, g1, b1, W2, a2, g2, b2, We,
           Wm1, am1, gm1, bm1, Wm2, am2, gm2, bm2, Wc,
           patch_src, patch_dst, patch_seg, mesh_src, mesh_dst):
    i32 = jnp.int32
    f32 = jnp.float32

    restride = lambda v: jnp.pad(
        jnp.pad(v.reshape(NWORK, EPW_P), ((0, 0), (0, 1))).reshape(-1), (0, WIN))
    psrc = restride(patch_src.astype(i32))
    pdst = restride(patch_dst.astype(i32))
    pew = restride(patch_ew.astype(f32))
    restride_m = lambda v: jnp.pad(
        v.reshape(NWORK, EPW_M), ((0, 0), (0, STRIDE_M - EPW_M))).reshape(-1)
    msrc = restride_m(mesh_src.astype(i32))
    mdst = restride_m(mesh_dst.astype(i32))
    mewp = restride_m(mesh_ew.astype(f32))
    z1 = jnp.zeros((STOT,), f32)
    z2 = jnp.zeros((NPR, HM), f32)

    stats = _sc_stats(psrc, pdst, pew, msrc, mdst, z1)

    a_pad = jnp.pad(stats[:, OFF_A:OFF_A + 100 * NP].reshape(2, 100, NP),
                    ((0, 0), (0, 4), (0, NPAD - NP)))
    pdo = jnp.pad(stats[:, OFF_PDO:OFF_PDO + PS * NP].reshape(2, PS, NP),
                  ((0, 0), (0, 16 - PS), (0, NPAD - NP)))
    pdi = jnp.pad(stats[:, OFF_PDI:OFF_PDI + PS * NP].reshape(2, PS, NP),
                  ((0, 0), (0, 16 - PS), (0, NPAD - NP)))
    mdo_raw = stats[:, OFF_MDO:OFF_MDO + NP]
    mdi_raw = stats[:, OFF_MDI:OFF_MDI + NP]
    mdo_p = jnp.pad(mdo_raw[:, None, :], ((0, 0), (0, 7), (0, NPAD - NP)))
    mdo_m = jnp.pad(mdo_raw[:, None, :], ((0, 0), (0, 7), (0, 0)))
    mdi_m = jnp.pad(mdi_raw[:, None, :], ((0, 0), (0, 7), (0, 0)))

    xs = jnp.pad(patch_feats.reshape(NP, PS, IN).transpose(1, 2, 0),
                 ((0, 0), (0, 0), (0, NPAD - NP)))
    col = lambda v: jnp.broadcast_to(v[:, None], (v.shape[0], 8))

    roT, hp1T = _tc_patch(xs, a_pad, pdo, pdi, mdo_p,
                          W1.T, W2.T, We.T, Wm1.T,
                          col(a1), col(g1), col(b1),
                          col(a2), col(g2), col(b2))

    mdi_c = jnp.pad(mdi_raw.T, ((0, 0), (0, 6)))
    mdo_c = jnp.pad(mdo_raw.T, ((0, 0), (0, 6)))
    row = lambda v: jnp.broadcast_to(v[None, :], (8, v.shape[0]))

    hp1 = hp1T[:, :NP].T
    agg1 = _sc_mesh(hp1, msrc, mdst, mewp, z2)[:, :NP]

    hp2, ra = _tc_mesh1(agg1, mdi_c, mdo_c, Wm2,
                        row(am1), row(gm1), row(bm1))

    agg2 = _sc_mesh(hp2, msrc, mdst, mewp, z2)[:, :NP]

    out = _tc_mesh2(agg2, mdi_c, row(am2), row(gm2), row(bm2), ra, Wc)
    return out[0:1, :]


# speedup vs baseline: 1.4435x; 1.4403x over previous
import functools

import jax
import jax.numpy as jnp
from jax import lax
from jax.experimental import pallas as pl
from jax.experimental.pallas import tpu as pltpu
from jax.experimental.pallas import tpu_sc as plsc

NP = 10000
PS = 10
NN = 100000
EP = 300000
EM = 320000
IN = 128
H1 = 512
H2 = 128
RD = 128
HM = 128
OF = 16

PT = 128
NPAD = 10240
NT = NPAD // PT

NWORK = 32
EPW_P = EP // NWORK
STRIDE_P = EPW_P + 1
EPW_M = EM // NWORK
WIN = 128
NWIN_P = -(-EPW_P // WIN)
NWIN_M = -(-EPW_M // WIN)
NWIN_M2 = 80

OFF_A = 0
OFF_PDO = 1_000_000
OFF_PDI = 1_100_000
OFF_MDO = 1_200_000
OFF_MDI = 1_210_000
STOT = 1_220_608

NPR = 10240
CHUNK = 80
NCHUNKP = 126
STRIDE_M = 10240

@functools.cache
def _sc_vector_mesh():
    return plsc.VectorSubcoreMesh(core_axis_name="c", subcore_axis_name="s")


def _leaky(x):
    return jnp.where(x > 0, x, 0.01 * x)


def _sc_stats_body(psrc, pdst, pew, msrc, mdst, zsrc, out, acc, *bufs):
    (sA, dA, wA, sB, dB, wB,
     aiA, oiA, iiA, wuA, ouA, aiB, oiB, iiB, wuB, ouB,
     siA, siB, ssA, ssB) = bufs
    cid = lax.axis_index("c")
    sid = lax.axis_index("s")
    wid = cid * 16 + sid
    seg = STOT // 16

    z0 = sid * seg
    pltpu.sync_copy(zsrc.at[pl.ds(z0, seg)], acc.at[pl.ds(z0, seg)])
    plsc.subcore_barrier()

    def loads(src_hbm, dst_hbm, ew_hbm, base, sb, db, wb, sem):
        pltpu.async_copy(src_hbm.at[pl.ds(base, WIN)], sb, sem)
        pltpu.async_copy(dst_hbm.at[pl.ds(base, WIN)], db, sem)
        if ew_hbm is not None:
            pltpu.async_copy(ew_hbm.at[pl.ds(base, WIN)], wb, sem)

    def wait_loads(src_hbm, sb, db, wb, sem, with_ew):
        n = 3 if with_ew else 2
        for _ in range(n):
            pltpu.make_async_copy(src_hbm.at[pl.ds(0, WIN)], sb, sem).wait()

    def patch_compute(w, sb, db, wb, ai, oi, ii, wu, ou):
        for g in range(WIN // 16):
            sl = pl.ds(g * 16, 16)
            sv = sb[sl]
            dv = db[sl]
            p = ((sv.astype(jnp.float32) + 0.5) * 0.1).astype(jnp.int32)
            j = sv - p * PS
            i = dv - p * PS
            iota = lax.iota(jnp.int32, 16)
            valid = (w * WIN + g * 16 + iota) < EPW_P
            ai[sl] = p * 100 + i * 10 + j
            oi[sl] = OFF_PDO + j * NP + p
            ii[sl] = OFF_PDI + i * NP + p
            wu[sl] = jnp.where(valid, wb[sl], 0.0)
            ou[sl] = jnp.where(valid, 1.0, 0.0)

    def patch_scatters(ai, oi, ii, wu, ou, sem):
        pltpu.async_copy(wu, acc.at[ai], sem, add=True)
        pltpu.async_copy(ou, acc.at[oi], sem, add=True)
        pltpu.async_copy(ou, acc.at[ii], sem, add=True)

    def wait_scatters(ai, wu, sem, n):
        for _ in range(n):
            pltpu.make_async_copy(wu, acc.at[ai], sem).wait()

    base_p = wid * STRIDE_P
    loads(psrc, pdst, pew, base_p, sA, dA, wA, siA)
    loads(psrc, pdst, pew, base_p + WIN, sB, dB, wB, siB)

    def patch_pair(u, _):
        w0 = 2 * u
        w1 = w0 + 1
        wait_loads(psrc, sA, dA, wA, siA, True)
        patch_compute(w0, sA, dA, wA, aiA, oiA, iiA, wuA, ouA)

        @pl.when(u + 1 < NWIN_P // 2)
        def _la():
            loads(psrc, pdst, pew, base_p + (w0 + 2) * WIN, sA, dA, wA, siA)

        @pl.when(u > 0)
        def _wa():
            wait_scatters(aiA, wuA, ssA, 3)
        patch_scatters(aiA, oiA, iiA, wuA, ouA, ssA)

        wait_loads(psrc, sB, dB, wB, siB, True)
        patch_compute(w1, sB, dB, wB, aiB, oiB, iiB, wuB, ouB)

        @pl.when(u + 1 < NWIN_P // 2)
        def _lb():
            loads(psrc, pdst, pew, base_p + (w1 + 2) * WIN, sB, dB, wB, siB)

        @pl.when(u > 0)
        def _wb():
            wait_scatters(aiB, wuB, ssB, 3)
        patch_scatters(aiB, oiB, iiB, wuB, ouB, ssB)
        return _

    lax.fori_loop(0, NWIN_P // 2, patch_pair, 0, unroll=False)
    wait_scatters(aiA, wuA, ssA, 3)
    wait_scatters(aiB, wuB, ssB, 3)

    def mesh_compute(w, sb, db, ai, oi, ou):
        for g in range(WIN // 16):
            sl = pl.ds(g * 16, 16)
            iota = lax.iota(jnp.int32, 16)
            valid = (w * WIN + g * 16 + iota) < EPW_M
            ai[sl] = OFF_MDO + sb[sl]
            oi[sl] = OFF_MDI + db[sl]
            ou[sl] = jnp.where(valid, 1.0, 0.0)

    def mesh_scatters(ai, oi, ou, sem):
        pltpu.async_copy(ou, acc.at[ai], sem, add=True)
        pltpu.async_copy(ou, acc.at[oi], sem, add=True)

    base_m = wid * STRIDE_M
    loads(msrc, mdst, None, base_m, sA, dA, wA, siA)
    loads(msrc, mdst, None, base_m + WIN, sB, dB, wB, siB)

    def mesh_pair(u, _):
        w0 = 2 * u
        w1 = w0 + 1
        wait_loads(msrc, sA, dA, wA, siA, False)
        mesh_compute(w0, sA, dA, aiA, oiA, ouA)

        @pl.when(u + 1 < NWIN_M2 // 2)
        def _la():
            loads(msrc, mdst, None, base_m + (w0 + 2) * WIN, sA, dA, wA, siA)

        @pl.when(u > 0)
        def _wa():
            wait_scatters(aiA, ouA, ssA, 2)
        mesh_scatters(aiA, oiA, ouA, ssA)

        wait_loads(msrc, sB, dB, wB, siB, False)
        mesh_compute(w1, sB, dB, aiB, oiB, ouB)

        @pl.when(u + 1 < NWIN_M2 // 2)
        def _lb():
            loads(msrc, mdst, None, base_m + (w1 + 2) * WIN, sB, dB, wB, siB)

        @pl.when(u > 0)
        def _wb():
            wait_scatters(aiB, ouB, ssB, 2)
        mesh_scatters(aiB, oiB, ouB, ssB)
        return _

    lax.fori_loop(0, NWIN_M2 // 2, mesh_pair, 0, unroll=False)
    wait_scatters(aiA, ouA, ssA, 2)
    wait_scatters(aiB, ouB, ssB, 2)

    plsc.subcore_barrier()
    o = sid * seg
    pltpu.sync_copy(acc.at[pl.ds(o, seg)], out.at[cid, pl.ds(o, seg)])


@jax.jit
def _sc_stats(psrc, pdst, pew, msrc, mdst, zsrc):
    ibuf = lambda: pltpu.VMEM((WIN,), jnp.int32)
    fbuf = lambda: pltpu.VMEM((WIN,), jnp.float32)
    return pl.kernel(
        _sc_stats_body,
        out_type=jax.ShapeDtypeStruct((2, STOT), jnp.float32),
        mesh=_sc_vector_mesh(),
        scratch_types=[
            pltpu.VMEM_SHARED((STOT,), jnp.float32),
            ibuf(), ibuf(), fbuf(), ibuf(), ibuf(), fbuf(),
            ibuf(), ibuf(), ibuf(), fbuf(), fbuf(),
            ibuf(), ibuf(), ibuf(), fbuf(), fbuf(),
            pltpu.SemaphoreType.DMA, pltpu.SemaphoreType.DMA,
            pltpu.SemaphoreType.DMA, pltpu.SemaphoreType.DMA,
        ],
    )(psrc, pdst, pew, msrc, mdst, zsrc)


def _sc_mesh_body(h, msrc, mdst, mew, zsrc, out, acc,
                  srcA, srcB, ewA, ewB, dfA, dfB, drA, drB, rowsA, rowsB,
                  siA, siB, sgA, sgB, ssA, ssB):
    cid = lax.axis_index("c")
    sid = lax.axis_index("s")
    wid = cid * 16 + sid
    rpw = NPR // 16

    r0 = sid * rpw
    pltpu.sync_copy(zsrc.at[pl.ds(r0, rpw)], acc.at[pl.ds(r0, rpw)])
    plsc.subcore_barrier()

    woff = wid * STRIDE_M

    def idx_dmas(t, src_b, ew_b, df_b, sem):
        o = woff + t * CHUNK
        pltpu.async_copy(msrc.at[pl.ds(o, CHUNK)], src_b, sem)
        pltpu.async_copy(mew.at[pl.ds(o, CHUNK)], ew_b, sem)
        pltpu.async_copy(mdst.at[pl.ds(o, CHUNK)], df_b, sem)

    def wait_idx(src_b, ew_b, df_b, sem):
        pltpu.make_async_copy(msrc.at[pl.ds(0, CHUNK)], src_b, sem).wait()
        pltpu.make_async_copy(mew.at[pl.ds(0, CHUNK)], ew_b, sem).wait()
        pltpu.make_async_copy(mdst.at[pl.ds(0, CHUNK)], df_b, sem).wait()

    def scale(ew_b, rows):
        for g in range(CHUNK // 16):
            wv = ew_b[pl.ds(g * 16, 16)]
            for l in range(16):
                e = g * 16 + l
                w = wv[l]
                for k in range(HM // 16):
                    sl = pl.ds(k * 16, 16)
                    rows[e, sl] = rows[e, sl] * w

    def fill_row(df_b, dr_b):
        for g in range(CHUNK // 16):
            sl = pl.ds(g * 16, 16)
            dr_b[0, sl] = df_b[sl]

    idx_dmas(0, srcA, ewA, dfA, siA)
    idx_dmas(1, srcB, ewB, dfB, siB)
    wait_idx(srcA, ewA, dfA, siA)
    pltpu.async_copy(h.at[srcA], rowsA, sgA)
    wait_idx(srcB, ewB, dfB, siB)
    pltpu.async_copy(h.at[srcB], rowsB, sgB)

    def pair(u, _):
        t0 = 2 * u
        t1 = t0 + 1

        pltpu.make_async_copy(h.at[srcA], rowsA, sgA).wait()
        scale(ewA, rowsA)

        @pl.when(u > 0)
        def _wsa():
            pltpu.make_async_copy(rowsA, acc.at[drA.at[0]], ssA).wait()
        fill_row(dfA, drA)
        pltpu.async_copy(rowsA, acc.at[drA.at[0]], ssA, add=True)

        @pl.when(t0 + 2 < NCHUNKP)
        def _ia():
            idx_dmas(t0 + 2, srcA, ewA, dfA, siA)

        pltpu.make_async_copy(h.at[srcB], rowsB, sgB).wait()
        scale(ewB, rowsB)

        @pl.when(u > 0)
        def _wsb():
            pltpu.make_async_copy(rowsB, acc.at[drB.at[0]], ssB).wait()
        fill_row(dfB, drB)
        pltpu.async_copy(rowsB, acc.at[drB.at[0]], ssB, add=True)

        @pl.when(t1 + 2 < NCHUNKP)
        def _ib():
            idx_dmas(t1 + 2, srcB, ewB, dfB, siB)

        @pl.when(t0 + 2 < NCHUNKP)
        def _ga():
            wait_idx(srcA, ewA, dfA, siA)
            pltpu.async_copy(h.at[srcA], rowsA, sgA)

        @pl.when(t1 + 2 < NCHUNKP)
        def _gb():
            wait_idx(srcB, ewB, dfB, siB)
            pltpu.async_copy(h.at[srcB], rowsB, sgB)

        return _

    lax.fori_loop(0, NCHUNKP // 2, pair, 0, unroll=False)

    pltpu.make_async_copy(rowsA, acc.at[drA.at[0]], ssA).wait()
    pltpu.make_async_copy(rowsB, acc.at[drB.at[0]], ssB).wait()

    plsc.subcore_barrier()
    pltpu.sync_copy(acc.at[pl.ds(r0, rpw)], out.at[cid, pl.ds(r0, rpw)])


@jax.jit
def _sc_mesh(h, msrc, mdst, mew, zsrc):
    return pl.kernel(
        _sc_mesh_body,
        out_type=jax.ShapeDtypeStruct((2, NPR, HM), jnp.float32),
        mesh=_sc_vector_mesh(),
        scratch_types=[
            pltpu.VMEM_SHARED((NPR, HM), jnp.float32),
            pltpu.VMEM((CHUNK,), jnp.int32),
            pltpu.VMEM((CHUNK,), jnp.int32),
            pltpu.VMEM((CHUNK,), jnp.float32),
            pltpu.VMEM((CHUNK,), jnp.float32),
            pltpu.VMEM((CHUNK,), jnp.int32),
            pltpu.VMEM((CHUNK,), jnp.int32),
            pltpu.VMEM((1, CHUNK), jnp.int32),
            pltpu.VMEM((1, CHUNK), jnp.int32),
            pltpu.VMEM((CHUNK, HM), jnp.float32),
            pltpu.VMEM((CHUNK, HM), jnp.float32),
            pltpu.SemaphoreType.DMA,
            pltpu.SemaphoreType.DMA,
            pltpu.SemaphoreType.DMA,
            pltpu.SemaphoreType.DMA,
            pltpu.SemaphoreType.DMA,
            pltpu.SemaphoreType.DMA,
        ],
    )(h, msrc, mdst, mew, zsrc)


def _tc_patch_body(xs_ref, a_ref, pdo_ref, pdi_ref, mdo_ref,
                   w1t_ref, w2t_ref, wet_ref, wm1t_ref,
                   a1_ref, g1_ref, b1_ref, a2_ref, g2_ref, b2_ref,
                   ro_ref, hp_ref):
    f32 = jnp.float32
    A2 = a_ref[0] + a_ref[1]
    do = lax.rsqrt(jnp.maximum(pdo_ref[0] + pdo_ref[1], 1.0))
    di = lax.rsqrt(jnp.maximum(pdi_ref[0] + pdi_ref[1], 1.0))
    xs = [xs_ref[j] for j in range(PS)]

    r0 = sum(xs) * (1.0 / PS)

    wA = [[A2[i * PS + j:i * PS + j + 1, :] * do[j:j + 1, :]
           for j in range(PS)] for i in range(PS)]

    def mix(vs):
        outs = []
        for i in range(PS):
            acc = wA[i][0] * vs[0]
            for j in range(1, PS):
                acc = acc + wA[i][j] * vs[j]
            outs.append(di[i:i + 1, :] * acc)
        return outs

    def gnorm(hs, ac, gc, bc):
        m = sum(hs) * (1.0 / PS)
        sub = [h - ac * m for h in hs]
        var = sum(s * s for s in sub) * (1.0 / PS)
        inv = lax.rsqrt(var + 1e-5)
        return [_leaky(gc * s * inv + bc) for s in sub]

    mixed = mix(xs)
    h1 = [jnp.dot(w1t_ref[...], m, preferred_element_type=f32) for m in mixed]
    h1n = gnorm(h1, a1_ref[:, 0:1], g1_ref[:, 0:1], b1_ref[:, 0:1])
    r1 = sum(h1n) * (1.0 / PS)

    t = [jnp.dot(w2t_ref[...], h, preferred_element_type=f32) for h in h1n]
    mixed2 = mix(t)
    h2n = gnorm(mixed2, a2_ref[:, 0:1], g2_ref[:, 0:1], b2_ref[:, 0:1])
    r2 = sum(h2n) * (1.0 / PS)

    cat = jnp.concatenate([r0, r1, r2], axis=0)
    emb = jnp.dot(wet_ref[...], cat, preferred_element_type=f32)
    mu = jnp.mean(emb, axis=0, keepdims=True)
    var = jnp.mean((emb - mu) ** 2, axis=0, keepdims=True)
    ro = _leaky((emb - mu) * lax.rsqrt(var + 1e-5))
    ro_ref[...] = ro

    mdo_r = lax.rsqrt(jnp.maximum(mdo_ref[0, 0:1, :] + mdo_ref[1, 0:1, :], 1.0))
    hp_ref[...] = jnp.dot(wm1t_ref[...], ro, preferred_element_type=f32) * mdo_r


@jax.jit
def _tc_patch(xs, a_pad, pdo, pdi, mdo, w1t, w2t, wet, wm1t,
              a1c, g1c, b1c, a2c, g2c, b2c):
    full = lambda s: pl.BlockSpec(s, lambda t: (0,) * len(s))
    tiled3 = lambda d0, d1: pl.BlockSpec((d0, d1, PT), lambda t: (0, 0, t))
    return pl.pallas_call(
        _tc_patch_body,
        grid=(NT,),
        in_specs=[
            tiled3(PS, IN),
            tiled3(2, 104),
            tiled3(2, 16),
            tiled3(2, 16),
            tiled3(2, 8),
            full((H1, IN)), full((H2, H1)), full((RD, IN + H1 + H2)),
            full((HM, RD)),
            full((H1, 8)), full((H1, 8)), full((H1, 8)),
            full((H2, 8)), full((H2, 8)), full((H2, 8)),
        ],
        out_specs=[
            pl.BlockSpec((RD, PT), lambda t: (0, t)),
            pl.BlockSpec((HM, PT), lambda t: (0, t)),
        ],
        out_shape=[
            jax.ShapeDtypeStruct((RD, NPAD), jnp.float32),
            jax.ShapeDtypeStruct((HM, NPAD), jnp.float32),
        ],
        compiler_params=pltpu.CompilerParams(vmem_limit_bytes=100 * 1024 * 1024),
    )(xs, a_pad, pdo, pdi, mdo, w1t, w2t, wet, wm1t,
      a1c, g1c, b1c, a2c, g2c, b2c)


def _mesh_norm(x, ar, gr, br):
    mu = jnp.mean(x, axis=0, keepdims=True)
    sub = x - ar * mu
    var = jnp.mean(sub * sub, axis=0, keepdims=True)
    return _leaky(gr * sub * lax.rsqrt(var + 1e-5) + br)


def _tc_mesh1_body(agg_ref, mdi_ref, mdo_ref, wm2_ref, am_ref, gm_ref, bm_ref,
                   hp_ref, ra_ref):
    mdi = lax.rsqrt(jnp.maximum(mdi_ref[:, 0:1] + mdi_ref[:, 1:2], 1.0))
    x = (agg_ref[0] + agg_ref[1]) * mdi
    hm = _mesh_norm(x, am_ref[0:1, :], gm_ref[0:1, :], bm_ref[0:1, :])
    ra_ref[...] = jnp.broadcast_to(jnp.mean(hm, axis=0, keepdims=True), (8, HM))
    mdo = lax.rsqrt(jnp.maximum(mdo_ref[:, 0:1] + mdo_ref[:, 1:2], 1.0))
    hp_ref[...] = jnp.dot(hm, wm2_ref[...],
                          preferred_element_type=jnp.float32) * mdo


@jax.jit
def _tc_mesh1(agg, mdi, mdo, wm2, amr, gmr, bmr):
    full = lambda s: pl.BlockSpec(s, lambda: (0,) * len(s))
    return pl.pallas_call(
        _tc_mesh1_body,
        in_specs=[full((2, NP, HM)), full((NP, 8)), full((NP, 8)),
                  full((HM, HM)), full((8, HM)), full((8, HM)), full((8, HM))],
        out_specs=[full((NP, HM)), full((8, HM))],
        out_shape=[jax.ShapeDtypeStruct((NP, HM), jnp.float32),
                   jax.ShapeDtypeStruct((8, HM), jnp.float32)],
        compiler_params=pltpu.CompilerParams(vmem_limit_bytes=100 * 1024 * 1024),
    )(agg, mdi, mdo, wm2, amr, gmr, bmr)


def _tc_mesh2_body(agg_ref, mdi_ref, am_ref, gm_ref, bm_ref, ra_ref, wc_ref,
                   out_ref):
    mdi = lax.rsqrt(jnp.maximum(mdi_ref[:, 0:1] + mdi_ref[:, 1:2], 1.0))
    x = (agg_ref[0] + agg_ref[1]) * mdi
    hm = _mesh_norm(x, am_ref[0:1, :], gm_ref[0:1, :], bm_ref[0:1, :])
    rb = jnp.mean(hm, axis=0, keepdims=True)
    cat = jnp.concatenate([ra_ref[0:1, :], rb], axis=1)
    blk = jnp.broadcast_to(_leaky(cat), (8, 2 * HM))
    out_ref[...] = jnp.dot(blk, wc_ref[...], preferred_element_type=jnp.float32)


@jax.jit
def _tc_mesh2(agg, mdi, amr, gmr, bmr, ra, wc):
    full = lambda s: pl.BlockSpec(s, lambda: (0,) * len(s))
    return pl.pallas_call(
        _tc_mesh2_body,
        in_specs=[full((2, NP, HM)), full((NP, 8)),
                  full((8, HM)), full((8, HM)), full((8, HM)), full((8, HM)),
                  full((2 * HM, OF))],
        out_specs=full((8, OF)),
        out_shape=jax.ShapeDtypeStruct((8, OF), jnp.float32),
        compiler_params=pltpu.CompilerParams(vmem_limit_bytes=100 * 1024 * 1024),
    )(agg, mdi, amr, gmr, bmr, ra, wc)


def kernel(patch_feats, patch_ew, mesh_ew, W1, a1, g1, b1, W2, a2, g2, b2, We,
           Wm1, am1, gm1, bm1, Wm2, am2, gm2, bm2, Wc,
           patch_src, patch_dst, patch_seg, mesh_src, mesh_dst):
    i32 = jnp.int32
    f32 = jnp.float32

    restride = lambda v: jnp.pad(
        jnp.pad(v.reshape(NWORK, EPW_P), ((0, 0), (0, 1))).reshape(-1), (0, WIN))
    psrc = restride(patch_src.astype(i32))
    pdst = restride(patch_dst.astype(i32))
    pew = restride(patch_ew.astype(f32))
    restride_m = lambda v: jnp.pad(
        v.reshape(NWORK, EPW_M), ((0, 0), (0, STRIDE_M - EPW_M))).reshape(-1)
    msrc = restride_m(mesh_src.astype(i32))
    mdst = restride_m(mesh_dst.astype(i32))
    mewp = restride_m(mesh_ew.astype(f32))
    z1 = jnp.zeros((STOT,), f32)
    z2 = jnp.zeros((NPR, HM), f32)

    stats = _sc_stats(psrc, pdst, pew, msrc, mdst, z1)

    a_pad = jnp.pad(stats[:, OFF_A:OFF_A + 100 * NP].reshape(2, 100, NP),
                    ((0, 0), (0, 4), (0, NPAD - NP)))
    pdo = jnp.pad(stats[:, OFF_PDO:OFF_PDO + PS * NP].reshape(2, PS, NP),
                  ((0, 0), (0, 16 - PS), (0, NPAD - NP)))
    pdi = jnp.pad(stats[:, OFF_PDI:OFF_PDI + PS * NP].reshape(2, PS, NP),
                  ((0, 0), (0, 16 - PS), (0, NPAD - NP)))
    mdo_raw = stats[:, OFF_MDO:OFF_MDO + NP]
    mdi_raw = stats[:, OFF_MDI:OFF_MDI + NP]
    mdo_p = jnp.pad(mdo_raw[:, None, :], ((0, 0), (0, 7), (0, NPAD - NP)))
    mdo_m = jnp.pad(mdo_raw[:, None, :], ((0, 0), (0, 7), (0, 0)))
    mdi_m = jnp.pad(mdi_raw[:, None, :], ((0, 0), (0, 7), (0, 0)))

    xs = jnp.pad(patch_feats.reshape(NP, PS, IN).transpose(1, 2, 0),
                 ((0, 0), (0, 0), (0, NPAD - NP)))
    col = lambda v: jnp.broadcast_to(v[:, None], (v.shape[0], 8))

    roT, hp1T = _tc_patch(xs, a_pad, pdo, pdi, mdo_p,
                          W1.T, W2.T, We.T, Wm1.T,
                          col(a1), col(g1), col(b1),
                          col(a2), col(g2), col(b2))

    mdi_c = jnp.pad(mdi_raw.T, ((0, 0), (0, 6)))
    mdo_c = jnp.pad(mdo_raw.T, ((0, 0), (0, 6)))
    row = lambda v: jnp.broadcast_to(v[None, :], (8, v.shape[0]))

    hp1 = hp1T[:, :NP].T
    agg1 = _sc_mesh(hp1, msrc, mdst, mewp, z2)[:, :NP]

    hp2, ra = _tc_mesh1(agg1, mdi_c, mdo_c, Wm2,
                        row(am1), row(gm1), row(bm1))

    agg2 = _sc_mesh(hp2, msrc, mdst, mewp, z2)[:, :NP]

    out = _tc_mesh2(agg2, mdi_c, row(am2), row(gm2), row(bm2), ra, Wc)
    return out[0:1, :]
